# baseline (speedup 1.0000x reference)
import functools

import jax
import jax.numpy as jnp
from jax import lax
from jax.experimental import pallas as pl
from jax.experimental.pallas import tpu as pltpu
from jax.experimental.pallas import tpu_sc as plsc

NB = 2


def _make_gather(vocab, d, nj, ni):
  info = plsc.get_sparse_core_info()
  nw = info.num_cores * info.num_subcores
  assert ni // 128 == nw and d == 64
  mesh = plsc.VectorSubcoreMesh(core_axis_name="c", subcore_axis_name="s")

  @functools.partial(
      pl.kernel,
      mesh=mesh,
      out_type=jax.ShapeDtypeStruct((nj, d // 8, nw, 8, 128), jnp.float32),
      scratch_types=(
          [pltpu.VMEM((nj, 128), jnp.int32)]
          + [pltpu.VMEM((128,), jnp.int32)] * NB
          + [pltpu.VMEM((128, 129), jnp.float32)] * NB
          + [pltpu.VMEM((d // 8, 8, 129), jnp.float32)] * NB
          + [pltpu.SemaphoreType.DMA] * (2 * NB)
      ),
      compiler_params=pltpu.CompilerParams(
          use_tc_tiling_on_sc=True, needs_layout_passes=False),
  )
  def gather(table2_hbm, xt_hbm, out_hbm, idx_v, *bufs):
    wbuf = bufs[:NB]
    rows = bufs[NB:2 * NB]
    obuf = bufs[2 * NB:3 * NB]
    gsem = bufs[3 * NB:4 * NB]
    osem = bufs[4 * NB:]
    wid = lax.axis_index("s") * info.num_cores + lax.axis_index("c")
    iota16 = lax.iota(jnp.int32, 16)
    drv = [(db * 16 + iota16) >> 3 for db in range(4)]
    dsv = [(db * 16 + iota16) & 7 for db in range(4)]

    pltpu.sync_copy(xt_hbm.at[:, pl.ds(wid * 128, 128)], idx_v)

    def issue_gather(j, s):
      for ib in range(8):
        wbuf[s][pl.ds(ib * 16, 16)] = idx_v[j, pl.ds(ib * 16, 16)] >> 1
      pltpu.async_copy(
          table2_hbm.at[wbuf[s]], rows[s].at[:, pl.ds(0, 128)], gsem[s])

    def wait_gather(j, s):
      pltpu.make_async_copy(
          table2_hbm.at[wbuf[s]], rows[s].at[:, pl.ds(0, 128)],
          gsem[s]).wait()

    def issue_out(j, s):
      pltpu.async_copy(
          obuf[s].at[:, :, pl.ds(0, 128)], out_hbm.at[j, :, wid], osem[s])

    def wait_out(j, s):
      pltpu.make_async_copy(
          obuf[s].at[:, :, pl.ds(0, 128)], out_hbm.at[j, :, wid],
          osem[s]).wait()

    def transpose(j, s):
      r, o = rows[s], obuf[s]
      pcol = [(idx_v[j, pl.ds(bb * 16, 16)] & 1) * 64 for bb in range(8)]
      rowv = [iota16 + bb * 16 for bb in range(8)]

      def dloop(dd, carry):
        dr = dd >> 3
        ds_ = dd & 7
        for bb in range(8):
          v = plsc.load_gather(r, [rowv[bb], pcol[bb] + dd])
          o[dr, ds_, pl.ds(bb * 16, 16)] = v
        return carry

      lax.fori_loop(0, 64, dloop, 0)

    for j in range(NB):
      issue_gather(j, j)

    for j in range(NB):
      wait_gather(j, j)
      transpose(j, j)
      issue_out(j, j)
      issue_gather(j + NB, j)

    def body(g, carry):
      for s in range(NB):
        j = g * NB + s
        wait_gather(j, s)
        wait_out(j - NB, s)
        transpose(j, s)
        issue_out(j, s)
        issue_gather(j + NB, s)
      return carry

    lax.fori_loop(1, nj // NB - 1, body, 0)

    i0 = nj - NB
    for s in range(NB):
      j = i0 + s
      wait_gather(j, s)
      wait_out(j - NB, s)
      transpose(j, s)
      issue_out(j, s)
    for s in range(NB):
      wait_out(i0 + s, s)

  return gather


def kernel(x, table):
  b, t = x.shape
  vocab, d = table.shape
  xt = x.T.astype(jnp.int32)
  table2 = table.reshape(vocab // 2, 2 * d)
  out5 = _make_gather(vocab, d, t, b)(table2, xt)
  return out5.transpose(2, 4, 0, 1, 3).reshape(b, t, d)


# speedup vs baseline: 1.4499x; 1.4499x over previous
import functools

import jax
import jax.numpy as jnp
from jax import lax
from jax.experimental import pallas as pl
from jax.experimental.pallas import tpu as pltpu
from jax.experimental.pallas import tpu_sc as plsc

NB = 2


def _make_gather(vocab, d, nj, ni):
  info = plsc.get_sparse_core_info()
  nw = info.num_cores * info.num_subcores
  assert ni // 128 == nw and d == 64
  mesh = plsc.VectorSubcoreMesh(core_axis_name="c", subcore_axis_name="s")

  @functools.partial(
      pl.kernel,
      mesh=mesh,
      out_type=jax.ShapeDtypeStruct((nj, d // 8, nw, 8, 128), jnp.float32),
      scratch_types=(
          [pltpu.VMEM((nj, 128), jnp.int32)]
          + [pltpu.VMEM((128,), jnp.int32)] * NB
          + [pltpu.VMEM((128, 129), jnp.float32)] * NB
          + [pltpu.VMEM((d // 8, 8, 129), jnp.float32)] * NB
          + [pltpu.SemaphoreType.DMA] * (2 * NB)
      ),
      compiler_params=pltpu.CompilerParams(
          use_tc_tiling_on_sc=True, needs_layout_passes=False),
  )
  def gather(table2_hbm, xt_hbm, out_hbm, idx_v, *bufs):
    wbuf = bufs[:NB]
    rows = bufs[NB:2 * NB]
    obuf = bufs[2 * NB:3 * NB]
    gsem = bufs[3 * NB:4 * NB]
    osem = bufs[4 * NB:]
    wid = lax.axis_index("s") * info.num_cores + lax.axis_index("c")
    iota16 = lax.iota(jnp.int32, 16)
    drv = [(db * 16 + iota16) >> 3 for db in range(4)]
    dsv = [(db * 16 + iota16) & 7 for db in range(4)]

    pltpu.sync_copy(xt_hbm.at[:, pl.ds(wid * 128, 128)], idx_v)

    def issue_gather(j, s):
      for ib in range(8):
        wbuf[s][pl.ds(ib * 16, 16)] = idx_v[j, pl.ds(ib * 16, 16)] >> 1
      pltpu.async_copy(
          table2_hbm.at[wbuf[s]], rows[s].at[:, pl.ds(0, 128)], gsem[s])

    def wait_gather(j, s):
      pltpu.make_async_copy(
          table2_hbm.at[wbuf[s]], rows[s].at[:, pl.ds(0, 128)],
          gsem[s]).wait()

    def issue_out(j, s):
      pltpu.async_copy(
          obuf[s].at[:, :, pl.ds(0, 128)], out_hbm.at[j, :, wid], osem[s])

    def wait_out(j, s):
      pltpu.make_async_copy(
          obuf[s].at[:, :, pl.ds(0, 128)], out_hbm.at[j, :, wid],
          osem[s]).wait()

    def transpose(j, s):
      r, o = rows[s], obuf[s]
      pcol = [(idx_v[j, pl.ds(bb * 16, 16)] & 1) * 64 for bb in range(8)]
      rowv = [iota16 + bb * 16 for bb in range(8)]

      @plsc.parallel_loop(0, 64, unroll=4)
      def dloop(dd):
        dr = dd >> 3
        ds_ = dd & 7
        for bb in range(8):
          v = plsc.load_gather(r, [rowv[bb], pcol[bb] + dd])
          o[dr, ds_, pl.ds(bb * 16, 16)] = v

    for j in range(NB):
      issue_gather(j, j)

    for j in range(NB):
      wait_gather(j, j)
      transpose(j, j)
      issue_out(j, j)
      issue_gather(j + NB, j)

    def body(g, carry):
      for s in range(NB):
        j = g * NB + s
        wait_gather(j, s)
        wait_out(j - NB, s)
        transpose(j, s)
        issue_out(j, s)
        issue_gather(j + NB, s)
      return carry

    lax.fori_loop(1, nj // NB - 1, body, 0)

    i0 = nj - NB
    for s in range(NB):
      j = i0 + s
      wait_gather(j, s)
      wait_out(j - NB, s)
      transpose(j, s)
      issue_out(j, s)
    for s in range(NB):
      wait_out(i0 + s, s)

  return gather


def kernel(x, table):
  b, t = x.shape
  vocab, d = table.shape
  xt = x.T.astype(jnp.int32)
  table2 = table.reshape(vocab // 2, 2 * d)
  out5 = _make_gather(vocab, d, t, b)(table2, xt)
  return out5.transpose(2, 4, 0, 1, 3).reshape(b, t, d)


# speedup vs baseline: 1.5334x; 1.0576x over previous
import functools

import jax
import jax.numpy as jnp
from jax import lax
from jax.experimental import pallas as pl
from jax.experimental.pallas import tpu as pltpu
from jax.experimental.pallas import tpu_sc as plsc

NB = 2


def _make_gather(vocab, d, nj, ni):
  info = plsc.get_sparse_core_info()
  nw = info.num_cores * info.num_subcores
  assert ni // 128 == nw and d == 64
  mesh = plsc.VectorSubcoreMesh(core_axis_name="c", subcore_axis_name="s")

  @functools.partial(
      pl.kernel,
      mesh=mesh,
      out_type=jax.ShapeDtypeStruct((nj, d // 8, nw, 8, 128), jnp.float32),
      scratch_types=(
          [pltpu.VMEM((nj, 128), jnp.int32)]
          + [pltpu.VMEM((128,), jnp.int32)] * NB
          + [pltpu.VMEM((128, 129), jnp.float32)] * NB
          + [pltpu.VMEM((d // 8, 8, 129), jnp.float32)] * NB
          + [pltpu.SemaphoreType.DMA] * (2 * NB)
      ),
      compiler_params=pltpu.CompilerParams(
          use_tc_tiling_on_sc=True, needs_layout_passes=False),
  )
  def gather(table2_hbm, xt_hbm, out_hbm, idx_v, *bufs):
    wbuf = bufs[:NB]
    rows = bufs[NB:2 * NB]
    obuf = bufs[2 * NB:3 * NB]
    gsem = bufs[3 * NB:4 * NB]
    osem = bufs[4 * NB:]
    wid = lax.axis_index("s") * info.num_cores + lax.axis_index("c")
    iota16 = lax.iota(jnp.int32, 16)
    drv = [(db * 16 + iota16) >> 3 for db in range(4)]
    dsv = [(db * 16 + iota16) & 7 for db in range(4)]

    pltpu.sync_copy(xt_hbm.at[:, pl.ds(wid * 128, 128)], idx_v)

    def issue_gather(j, s):
      pltpu.async_copy(
          table2_hbm.at[idx_v.at[j]], rows[s].at[:, pl.ds(0, 128)], gsem[s])

    def wait_gather(j, s):
      pltpu.make_async_copy(
          table2_hbm.at[idx_v.at[j]], rows[s].at[:, pl.ds(0, 128)],
          gsem[s]).wait()

    def issue_out(j, s):
      pltpu.async_copy(
          obuf[s].at[:, :, pl.ds(0, 128)], out_hbm.at[j, :, wid], osem[s])

    def wait_out(j, s):
      pltpu.make_async_copy(
          obuf[s].at[:, :, pl.ds(0, 128)], out_hbm.at[j, :, wid],
          osem[s]).wait()

    def transpose(j, s):
      r, o = rows[s], obuf[s]
      rowv = [iota16 + bb * 16 for bb in range(8)]
      zero16 = iota16 * 0

      @plsc.parallel_loop(0, 64, unroll=8)
      def dloop(dd):
        dr = dd >> 3
        ds_ = dd & 7
        cvec = zero16 + dd
        for bb in range(8):
          v = plsc.load_gather(r, [rowv[bb], cvec])
          o[dr, ds_, pl.ds(bb * 16, 16)] = v

    for j in range(NB):
      issue_gather(j, j)

    for j in range(NB):
      wait_gather(j, j)
      transpose(j, j)
      issue_out(j, j)
      issue_gather(j + NB, j)

    def body(g, carry):
      for s in range(NB):
        j = g * NB + s
        wait_gather(j, s)
        wait_out(j - NB, s)
        transpose(j, s)
        issue_out(j, s)
        issue_gather(j + NB, s)
      return carry

    lax.fori_loop(1, nj // NB - 1, body, 0)

    i0 = nj - NB
    for s in range(NB):
      j = i0 + s
      wait_gather(j, s)
      wait_out(j - NB, s)
      transpose(j, s)
      issue_out(j, s)
    for s in range(NB):
      wait_out(i0 + s, s)

  return gather


def kernel(x, table):
  b, t = x.shape
  vocab, d = table.shape
  xt = x.T.astype(jnp.int32)
  table2 = jnp.pad(table, ((0, 0), (0, d)))
  out5 = _make_gather(vocab, d, t, b)(table2, xt)
  return out5.transpose(2, 4, 0, 1, 3).reshape(b, t, d)
